# Initial kernel scaffold; baseline (speedup 1.0000x reference)
import jax
import jax.numpy as jnp
from jax.experimental import pallas as pl


def kernel(x, edge_index, W_l1, b_l1, W_r1, W_l2, b_l2, W_r2):
    raise NotImplementedError("write your pallas kernel here")


# speedup vs baseline: 6.1917x; 6.1917x over previous
import jax
import jax.numpy as jnp
from jax import lax
from jax.experimental import pallas as pl
from jax.experimental.pallas import tpu as pltpu
from jax.experimental.pallas import tpu_sc as plsc

N = 10000
D = 128
E = 320000

NC = 2
NS = 16
NW = NC * NS
CHUNK = 128
NCHUNKS = E // CHUNK
FULL = NCHUNKS // NW
REM = NCHUNKS - FULL * NW
RPSP = 632
NPAD = NS * RPSP
CPS = 1024
CPAD = 10 * CPS

_mesh = plsc.VectorSubcoreMesh(core_axis_name="c", subcore_axis_name="s")


def _seg_body(src_hbm, dst_hbm, x_hbm, s_out, c_out,
              src_v, dst_v, rows_v, ones_v, zrow_v, zcnt_v,
              acc_sh, cnt_sh, sem):
    cid = lax.axis_index("c")
    sid = lax.axis_index("s")
    wid = sid * NC + cid

    zero16 = jnp.zeros((16,), jnp.float32)
    one16 = jnp.ones((16,), jnp.float32)

    def _zr(i, carry):
        for j in range(D // 16):
            zrow_v[i, pl.ds(16 * j, 16)] = zero16
        return carry
    lax.fori_loop(0, 79, _zr, 0)

    def _zc(i, carry):
        zcnt_v[pl.ds(16 * i, 16)] = zero16
        return carry
    lax.fori_loop(0, CPS // 16, _zc, 0)

    for j in range(CHUNK // 16):
        ones_v[pl.ds(16 * j, 16)] = one16

    for j in range(8):
        pltpu.sync_copy(zrow_v, acc_sh.at[pl.ds(sid * RPSP + j * 79, 79)])

    @pl.when(sid < 10)
    def _():
        pltpu.sync_copy(zcnt_v, cnt_sh.at[pl.ds(sid * CPS, CPS)])

    plsc.subcore_barrier()

    def _do_chunk(g):
        pltpu.sync_copy(src_hbm.at[g], src_v)
        pltpu.sync_copy(dst_hbm.at[g], dst_v)
        pltpu.async_copy(x_hbm.at[src_v], rows_v, sem).wait()
        pltpu.sync_copy(rows_v, acc_sh.at[dst_v], add=True)
        pltpu.sync_copy(ones_v, cnt_sh.at[dst_v], add=True)

    def _body(ci, carry):
        _do_chunk(ci * NW + wid)
        return carry
    lax.fori_loop(0, FULL, _body, 0)

    @pl.when(wid < REM)
    def _():
        _do_chunk(FULL * NW + wid)

    plsc.subcore_barrier()

    pltpu.sync_copy(acc_sh.at[pl.ds(sid * RPSP, RPSP)],
                    s_out.at[cid * NS + sid])

    @pl.when(sid < 10)
    def _():
        pltpu.sync_copy(cnt_sh.at[pl.ds(sid * CPS, CPS)],
                        c_out.at[cid * 10 + sid])


_seg_call = pl.kernel(
    _seg_body,
    out_type=(
        jax.ShapeDtypeStruct((NC * NS, RPSP, D), jnp.float32),
        jax.ShapeDtypeStruct((NC * 10, CPS), jnp.float32),
    ),
    mesh=_mesh,
    scratch_types=[
        pltpu.VMEM((CHUNK,), jnp.int32),
        pltpu.VMEM((CHUNK,), jnp.int32),
        pltpu.VMEM((CHUNK, D), jnp.float32),
        pltpu.VMEM((CHUNK,), jnp.float32),
        pltpu.VMEM((79, D), jnp.float32),
        pltpu.VMEM((CPS,), jnp.float32),
        pltpu.VMEM_SHARED((NPAD, D), jnp.float32),
        pltpu.VMEM_SHARED((CPAD,), jnp.float32),
        pltpu.SemaphoreType.DMA,
    ],
)


BN = 1000


def _dense_kernel(s_ref, c_ref, x_ref, wl_ref, b_ref, wr_ref, o_ref):
    s = s_ref[0] + s_ref[1]
    cnt = c_ref[..., 0:1] + c_ref[..., 1:2]
    agg = s / jnp.maximum(cnt, 1.0)
    out = lax.dot_general(agg, wl_ref[...], (((1,), (1,)), ((), ())),
                          preferred_element_type=jnp.float32)
    out = out + b_ref[...]
    out = out + lax.dot_general(x_ref[...], wr_ref[...], (((1,), (1,)), ((), ())),
                                preferred_element_type=jnp.float32)
    nrm = jnp.sqrt(jnp.sum(out * out, axis=1, keepdims=True))
    out = out / jnp.maximum(nrm, 1e-12)
    o_ref[...] = jnp.maximum(out, 0.0)


def _dense(s2, c2t, x, W_l, b, W_r):
    return pl.pallas_call(
        _dense_kernel,
        grid=(N // BN,),
        in_specs=[
            pl.BlockSpec((2, BN, D), lambda i: (0, i, 0)),
            pl.BlockSpec((BN, 2), lambda i: (i, 0)),
            pl.BlockSpec((BN, D), lambda i: (i, 0)),
            pl.BlockSpec((D, D), lambda i: (0, 0)),
            pl.BlockSpec((1, D), lambda i: (0, 0)),
            pl.BlockSpec((D, D), lambda i: (0, 0)),
        ],
        out_specs=pl.BlockSpec((BN, D), lambda i: (i, 0)),
        out_shape=jax.ShapeDtypeStruct((N, D), jnp.float32),
    )(s2, c2t, x, W_l, b, W_r)


def kernel(x, edge_index, W_l1, b_l1, W_r1, W_l2, b_l2, W_r2):
    src = edge_index[0].reshape(NCHUNKS, CHUNK)
    dst = edge_index[1].reshape(NCHUNKS, CHUNK)
    b1 = b_l1.reshape(1, D)
    b2 = b_l2.reshape(1, D)

    s1, c1 = _seg_call(src, dst, x)
    s1 = s1.reshape(NC, NPAD, D)
    c1t = c1.reshape(NC, CPAD).T
    h = _dense(s1, c1t, x, W_l1, b1, W_r1)

    s2, c2 = _seg_call(src, dst, h)
    s2 = s2.reshape(NC, NPAD, D)
    c2t = c2.reshape(NC, CPAD).T
    return _dense(s2, c2t, h, W_l2, b2, W_r2)


# speedup vs baseline: 7.7685x; 1.2547x over previous
import jax
import jax.numpy as jnp
from jax import lax
from jax.experimental import pallas as pl
from jax.experimental.pallas import tpu as pltpu
from jax.experimental.pallas import tpu_sc as plsc

N = 10000
D = 128
E = 320000

NC = 2
NS = 16
CHUNK = 128
CPT = 80
EPAD = NC * NS * CPT * CHUNK
PAD = EPAD - E
RPSP = 632
NPAD = NS * RPSP
CW = 128
ZTAIL = RPSP - 4 * CHUNK

_mesh = plsc.VectorSubcoreMesh(core_axis_name="c", subcore_axis_name="s")


def _seg_body(e_hbm, x_hbm, zr_hbm, s_out,
              idx0, idx1, r0, r1, acc_sh, g0, g1):
    idxs = [idx0, idx1]
    rows = [r0, r1]
    gsem = [g0, g1]

    cid = lax.axis_index("c")
    sid = lax.axis_index("s")
    eh = e_hbm.at[cid].at[sid]

    for j in range(4):
        pltpu.sync_copy(zr_hbm, acc_sh.at[pl.ds(sid * RPSP + j * CHUNK, CHUNK)])
    pltpu.sync_copy(zr_hbm.at[pl.ds(0, ZTAIL)],
                    acc_sh.at[pl.ds(sid * RPSP + 4 * CHUNK, ZTAIL)])

    plsc.subcore_barrier()

    def do(ci, b, last):
        if not last:
            pltpu.sync_copy(eh.at[ci + 1], idxs[1 - b])
        pltpu.make_async_copy(x_hbm.at[idxs[b].at[0]], rows[b],
                              gsem[b]).wait()
        if not last:
            pltpu.async_copy(x_hbm.at[idxs[1 - b].at[0]], rows[1 - b],
                             gsem[1 - b])
        pltpu.sync_copy(rows[b], acc_sh.at[idxs[b].at[1]], add=True)

    pltpu.sync_copy(eh.at[0], idx0)
    pltpu.async_copy(x_hbm.at[idx0.at[0]], r0, g0)

    def grp(g, carry):
        ci = 2 * g
        do(ci, 0, False)
        do(ci + 1, 1, False)
        return carry
    lax.fori_loop(0, (CPT - 2) // 2, grp, 0)
    do(CPT - 2, 0, False)
    do(CPT - 1, 1, True)

    plsc.subcore_barrier()

    pltpu.sync_copy(acc_sh.at[pl.ds(sid * RPSP, RPSP)],
                    s_out.at[cid * NS + sid])


_seg_call = pl.kernel(
    _seg_body,
    out_type=jax.ShapeDtypeStruct((NC * NS, RPSP, D), jnp.float32),
    mesh=_mesh,
    scratch_types=[
        pltpu.VMEM((2, CHUNK), jnp.int32),
        pltpu.VMEM((2, CHUNK), jnp.int32),
        pltpu.VMEM((CHUNK, D), jnp.float32),
        pltpu.VMEM((CHUNK, D), jnp.float32),
        pltpu.VMEM_SHARED((NPAD, D), jnp.float32),
    ] + [pltpu.SemaphoreType.DMA] * 2,
)


def _cnt_body(e_hbm, on_hbm, zc_hbm, c_out, idx0, idx1, ones_v, cnt_sh):
    idxs = [idx0, idx1]

    cid = lax.axis_index("c")
    sid = lax.axis_index("s")
    eh = e_hbm.at[cid].at[sid]

    for j in range(4):
        pltpu.sync_copy(zc_hbm, cnt_sh.at[pl.ds(sid * RPSP + j * CHUNK, CHUNK)])
    pltpu.sync_copy(zc_hbm.at[pl.ds(0, ZTAIL)],
                    cnt_sh.at[pl.ds(sid * RPSP + 4 * CHUNK, ZTAIL)])
    pltpu.sync_copy(on_hbm, ones_v)

    plsc.subcore_barrier()

    pltpu.sync_copy(eh.at[0], idx0)

    def do(ci, b, last):
        if not last:
            pltpu.sync_copy(eh.at[ci + 1], idxs[1 - b])
        pltpu.sync_copy(ones_v, cnt_sh.at[idxs[b].at[1]], add=True)

    def grp(g, carry):
        ci = 2 * g
        do(ci, 0, False)
        do(ci + 1, 1, False)
        return carry
    lax.fori_loop(0, (CPT - 2) // 2, grp, 0)
    do(CPT - 2, 0, False)
    do(CPT - 1, 1, True)

    plsc.subcore_barrier()

    pltpu.sync_copy(cnt_sh.at[pl.ds(sid * RPSP, RPSP)],
                    c_out.at[cid * NS + sid])


_cnt_call = pl.kernel(
    _cnt_body,
    out_type=jax.ShapeDtypeStruct((NC * NS, RPSP, CW), jnp.float32),
    mesh=_mesh,
    scratch_types=[
        pltpu.VMEM((2, CHUNK), jnp.int32),
        pltpu.VMEM((2, CHUNK), jnp.int32),
        pltpu.VMEM((CHUNK, CW), jnp.float32),
        pltpu.VMEM_SHARED((NPAD, CW), jnp.float32),
    ],
)


BN = 1000


def _dense_kernel(s_ref, c_ref, x_ref, wl_ref, b_ref, wr_ref, o_ref):
    cnt = jnp.maximum((c_ref[0] + c_ref[1])[:, 0:1], 1.0)
    agg = (s_ref[0] + s_ref[1]) / cnt
    out = lax.dot_general(agg, wl_ref[...], (((1,), (1,)), ((), ())),
                          preferred_element_type=jnp.float32)
    out += lax.dot_general(x_ref[...], wr_ref[...], (((1,), (1,)), ((), ())),
                           preferred_element_type=jnp.float32)
    out += b_ref[...]
    nrm = jnp.sqrt(jnp.sum(out * out, axis=1, keepdims=True))
    out = out / jnp.maximum(nrm, 1e-12)
    o_ref[...] = jnp.maximum(out, 0.0)


def _dense_call(s, c, x, W_l, b_l, W_r):
    return pl.pallas_call(
        _dense_kernel,
        grid=(N // BN,),
        in_specs=[
            pl.BlockSpec((NC, BN, D), lambda i: (0, i, 0)),
            pl.BlockSpec((NC, BN, CW), lambda i: (0, i, 0)),
            pl.BlockSpec((BN, D), lambda i: (i, 0)),
            pl.BlockSpec((D, D), lambda i: (0, 0)),
            pl.BlockSpec((1, D), lambda i: (0, 0)),
            pl.BlockSpec((D, D), lambda i: (0, 0)),
        ],
        out_specs=pl.BlockSpec((BN, D), lambda i: (i, 0)),
        out_shape=jax.ShapeDtypeStruct((N, D), jnp.float32),
    )(s, c, x, W_l, b_l, W_r)


def kernel(x, edge_index, W_l1, b_l1, W_r1, W_l2, b_l2, W_r2):
    ar = jnp.arange(PAD, dtype=jnp.int32)
    pad_src = (ar * 37) % N
    pad_dst = N + (ar % (NPAD - N))
    src = jnp.concatenate([edge_index[0], pad_src]).reshape(NC, NS, CPT, CHUNK)
    dst = jnp.concatenate([edge_index[1], pad_dst]).reshape(NC, NS, CPT, CHUNK)
    e = jnp.stack([src, dst], axis=3)

    zr = jnp.zeros((CHUNK, D), jnp.float32)
    on = jnp.ones((CHUNK, CW), jnp.float32)
    b1 = b_l1.reshape(1, D)
    b2 = b_l2.reshape(1, D)

    c1 = _cnt_call(e, on, zr)
    cc = c1.reshape(NC, NPAD, CW)
    s1 = _seg_call(e, x, zr).reshape(NC, NPAD, D)
    h = _dense_call(s1, cc, x, W_l1, b1, W_r1)

    s2 = _seg_call(e, h, zr).reshape(NC, NPAD, D)
    return _dense_call(s2, cc, h, W_l2, b2, W_r2)


# speedup vs baseline: 8.4358x; 1.0859x over previous
import jax
import jax.numpy as jnp
from jax import lax
from jax.experimental import pallas as pl
from jax.experimental.pallas import tpu as pltpu
from jax.experimental.pallas import tpu_sc as plsc

N = 10000
D = 128
E = 320000

NC = 2
NS = 16
CHUNK = 128
CPT = 80
EPAD = NC * NS * CPT * CHUNK
PAD = EPAD - E
RPSP = 632
NPAD = NS * RPSP
CW = 128
ZTAIL = RPSP - 4 * CHUNK

_mesh = plsc.VectorSubcoreMesh(core_axis_name="c", subcore_axis_name="s")


def _seg_body(e_hbm, x_hbm, zr_hbm, s_out,
              idx0, idx1, r0, r1, acc_sh, g0, g1, t0, t1):
    idxs = [idx0, idx1]
    rows = [r0, r1]
    gsem = [g0, g1]
    ssem = [t0, t1]

    cid = lax.axis_index("c")
    sid = lax.axis_index("s")
    eh = e_hbm.at[cid].at[sid]

    for j in range(4):
        pltpu.sync_copy(zr_hbm, acc_sh.at[pl.ds(sid * RPSP + j * CHUNK, CHUNK)])
    pltpu.sync_copy(zr_hbm.at[pl.ds(0, ZTAIL)],
                    acc_sh.at[pl.ds(sid * RPSP + 4 * CHUNK, ZTAIL)])

    plsc.subcore_barrier()

    def do(ci, b, first, last):
        if not first:
            pltpu.make_async_copy(rows[1 - b], acc_sh.at[idxs[1 - b].at[1]],
                                  ssem[1 - b]).wait()
        if not last:
            pltpu.sync_copy(eh.at[ci + 1], idxs[1 - b])
        pltpu.make_async_copy(x_hbm.at[idxs[b].at[0]], rows[b],
                              gsem[b]).wait()
        if not last:
            pltpu.async_copy(x_hbm.at[idxs[1 - b].at[0]], rows[1 - b],
                             gsem[1 - b])
        pltpu.async_copy(rows[b], acc_sh.at[idxs[b].at[1]], ssem[b],
                         add=True)

    pltpu.sync_copy(eh.at[0], idx0)
    pltpu.async_copy(x_hbm.at[idx0.at[0]], r0, g0)

    do(0, 0, True, False)

    def grp(g, carry):
        ci = 2 * g + 1
        do(ci, 1, False, False)
        do(ci + 1, 0, False, False)
        return carry
    lax.fori_loop(0, (CPT - 2) // 2, grp, 0)
    do(CPT - 1, 1, False, True)
    pltpu.make_async_copy(rows[1], acc_sh.at[idxs[1].at[1]], ssem[1]).wait()

    plsc.subcore_barrier()

    pltpu.sync_copy(acc_sh.at[pl.ds(sid * RPSP, RPSP)],
                    s_out.at[cid * NS + sid])


_seg_call = pl.kernel(
    _seg_body,
    out_type=jax.ShapeDtypeStruct((NC * NS, RPSP, D), jnp.float32),
    mesh=_mesh,
    scratch_types=[
        pltpu.VMEM((2, CHUNK), jnp.int32),
        pltpu.VMEM((2, CHUNK), jnp.int32),
        pltpu.VMEM((CHUNK, D), jnp.float32),
        pltpu.VMEM((CHUNK, D), jnp.float32),
        pltpu.VMEM_SHARED((NPAD, D), jnp.float32),
    ] + [pltpu.SemaphoreType.DMA] * 4,
)


def _cnt_body(e_hbm, on_hbm, zc_hbm, c_out, idx0, idx1, ones_v, cnt_sh,
              i0, i1):
    idxs = [idx0, idx1]
    isem = [i0, i1]

    cid = lax.axis_index("c")
    sid = lax.axis_index("s")
    eh = e_hbm.at[cid].at[sid]

    for j in range(4):
        pltpu.sync_copy(zc_hbm, cnt_sh.at[pl.ds(sid * RPSP + j * CHUNK, CHUNK)])
    pltpu.sync_copy(zc_hbm.at[pl.ds(0, ZTAIL)],
                    cnt_sh.at[pl.ds(sid * RPSP + 4 * CHUNK, ZTAIL)])
    pltpu.sync_copy(on_hbm, ones_v)

    plsc.subcore_barrier()

    pltpu.async_copy(eh.at[0], idx0, i0)
    pltpu.async_copy(eh.at[1], idx1, i1)

    def do(ci, b, last):
        pltpu.make_async_copy(eh.at[0], idxs[b], isem[b]).wait()
        pltpu.sync_copy(ones_v, cnt_sh.at[idxs[b].at[1]], add=True)
        if not last:
            pltpu.async_copy(eh.at[ci + 2], idxs[b], isem[b])

    def grp(g, carry):
        ci = 2 * g
        do(ci, 0, False)
        do(ci + 1, 1, False)
        return carry
    lax.fori_loop(0, (CPT - 2) // 2, grp, 0)
    do(CPT - 2, 0, True)
    do(CPT - 1, 1, True)

    plsc.subcore_barrier()

    pltpu.sync_copy(cnt_sh.at[pl.ds(sid * RPSP, RPSP)],
                    c_out.at[cid * NS + sid])


_cnt_call = pl.kernel(
    _cnt_body,
    out_type=jax.ShapeDtypeStruct((NC * NS, RPSP, CW), jnp.float32),
    mesh=_mesh,
    scratch_types=[
        pltpu.VMEM((2, CHUNK), jnp.int32),
        pltpu.VMEM((2, CHUNK), jnp.int32),
        pltpu.VMEM((CHUNK, CW), jnp.float32),
        pltpu.VMEM_SHARED((NPAD, CW), jnp.float32),
    ] + [pltpu.SemaphoreType.DMA] * 2,
)


BN = 1000


def _dense_kernel(s_ref, c_ref, x_ref, wl_ref, b_ref, wr_ref, o_ref):
    cnt = jnp.maximum((c_ref[0] + c_ref[1])[:, 0:1], 1.0)
    agg = (s_ref[0] + s_ref[1]) / cnt
    out = lax.dot_general(agg, wl_ref[...], (((1,), (1,)), ((), ())),
                          preferred_element_type=jnp.float32)
    out += lax.dot_general(x_ref[...], wr_ref[...], (((1,), (1,)), ((), ())),
                           preferred_element_type=jnp.float32)
    out += b_ref[...]
    nrm = jnp.sqrt(jnp.sum(out * out, axis=1, keepdims=True))
    out = out / jnp.maximum(nrm, 1e-12)
    o_ref[...] = jnp.maximum(out, 0.0)


def _dense_call(s, c, x, W_l, b_l, W_r):
    return pl.pallas_call(
        _dense_kernel,
        grid=(N // BN,),
        in_specs=[
            pl.BlockSpec((NC, BN, D), lambda i: (0, i, 0)),
            pl.BlockSpec((NC, BN, CW), lambda i: (0, i, 0)),
            pl.BlockSpec((BN, D), lambda i: (i, 0)),
            pl.BlockSpec((D, D), lambda i: (0, 0)),
            pl.BlockSpec((1, D), lambda i: (0, 0)),
            pl.BlockSpec((D, D), lambda i: (0, 0)),
        ],
        out_specs=pl.BlockSpec((BN, D), lambda i: (i, 0)),
        out_shape=jax.ShapeDtypeStruct((N, D), jnp.float32),
    )(s, c, x, W_l, b_l, W_r)


def kernel(x, edge_index, W_l1, b_l1, W_r1, W_l2, b_l2, W_r2):
    ar = jnp.arange(PAD, dtype=jnp.int32)
    pad_src = (ar * 37) % N
    pad_dst = N + (ar % (NPAD - N))
    src = jnp.concatenate([edge_index[0], pad_src]).reshape(NC, NS, CPT, CHUNK)
    dst = jnp.concatenate([edge_index[1], pad_dst]).reshape(NC, NS, CPT, CHUNK)
    e = jnp.stack([src, dst], axis=3)

    zr = jnp.zeros((CHUNK, D), jnp.float32)
    on = jnp.ones((CHUNK, CW), jnp.float32)
    b1 = b_l1.reshape(1, D)
    b2 = b_l2.reshape(1, D)

    c1 = _cnt_call(e, on, zr)
    cc = c1.reshape(NC, NPAD, CW)
    s1 = _seg_call(e, x, zr).reshape(NC, NPAD, D)
    h = _dense_call(s1, cc, x, W_l1, b1, W_r1)

    s2 = _seg_call(e, h, zr).reshape(NC, NPAD, D)
    return _dense_call(s2, cc, h, W_l2, b2, W_r2)
